# baseline (speedup 1.0000x reference)
import functools
import jax
import jax.numpy as jnp
from jax import lax
from jax.experimental import pallas as pl
from jax.experimental.pallas import tpu as pltpu
from jax.experimental.pallas import tpu_sc as plsc

D_EMB = 64
NC = 2
NS = 16
NW = NC * NS
XR = 4
SPLITS = ((0, 104), (104, 96))


def _body(nchunks, ncols, x_hbm, table_hbm, out_hbm,
          idx0, idx1, rows0, rows1,
          gsem0, gsem1, wsem0, wsem1, isem0, isem1):
    wid = lax.axis_index("s") * NC + lax.axis_index("c")
    row_base = wid * (nchunks * XR)
    last = nchunks - 1

    def fire_gathers(idx_v, rows_v, sem):
        for r in range(XR):
            for off, ln in SPLITS:
                pltpu.async_copy(table_hbm.at[idx_v.at[r, pl.ds(off, ln)]],
                                 rows_v.at[r, pl.ds(off, ln)], sem)

    def wait_gathers(idx_v, rows_v, sem):
        for r in range(XR):
            for off, ln in SPLITS:
                pltpu.make_async_copy(table_hbm.at[idx_v.at[r, pl.ds(off, ln)]],
                                      rows_v.at[r, pl.ds(off, ln)], sem).wait()

    def fire_idx(g, idx_v, sem):
        pltpu.async_copy(x_hbm.at[pl.ds(row_base + g * XR, XR)], idx_v, sem)

    def wait_idx(idx_v, sem):
        pltpu.make_async_copy(x_hbm.at[pl.ds(0, XR)], idx_v, sem).wait()

    def fire_write(g, rows_v, sem):
        pltpu.async_copy(rows_v,
                         out_hbm.at[pl.ds(row_base + g * XR, XR), :, pl.ds(0, 64)],
                         sem)

    def wait_write(rows_v, sem):
        pltpu.make_async_copy(rows_v,
                              out_hbm.at[pl.ds(0, XR), :, pl.ds(0, 64)],
                              sem).wait()

    pltpu.sync_copy(x_hbm.at[pl.ds(row_base, XR)], idx0)
    fire_gathers(idx0, rows0, gsem0)
    fire_idx(1, idx1, isem1)
    wait_idx(idx1, isem1)
    fire_gathers(idx1, rows1, gsem1)
    wait_gathers(idx0, rows0, gsem0)
    fire_write(0, rows0, wsem0)
    fire_idx(2, idx0, isem0)

    def body(o, carry):
        g = 2 * o
        wait_write(rows0, wsem0)
        wait_idx(idx0, isem0)
        fire_gathers(idx0, rows0, gsem0)
        wait_gathers(idx1, rows1, gsem1)
        fire_write(g - 1, rows1, wsem1)
        fire_idx(jnp.minimum(g + 1, last), idx1, isem1)
        wait_write(rows1, wsem1)
        wait_idx(idx1, isem1)
        fire_gathers(idx1, rows1, gsem1)
        wait_gathers(idx0, rows0, gsem0)
        fire_write(g, rows0, wsem0)
        fire_idx(jnp.minimum(g + 2, last), idx0, isem0)
        return carry

    lax.fori_loop(1, nchunks // 2, body, 0)

    wait_write(rows0, wsem0)
    wait_gathers(idx1, rows1, gsem1)
    fire_write(last, rows1, wsem1)
    wait_idx(idx0, isem0)
    wait_write(rows1, wsem1)


def kernel(x, table):
    B0, S = x.shape
    assert S == 200 and B0 % (NW * XR) == 0
    nchunks = B0 // (NW * XR)
    assert nchunks >= 2 and nchunks % 2 == 0

    mesh = plsc.VectorSubcoreMesh(core_axis_name="c", subcore_axis_name="s")
    run = pl.kernel(
        functools.partial(_body, nchunks, S),
        mesh=mesh,
        compiler_params=pltpu.CompilerParams(use_tc_tiling_on_sc=False),
        out_type=jax.ShapeDtypeStruct((B0, S, 2 * D_EMB), jnp.float32),
        scratch_types=[
            pltpu.VMEM((XR, S), jnp.int32),
            pltpu.VMEM((XR, S), jnp.int32),
            pltpu.VMEM((XR, S, D_EMB), jnp.float32),
            pltpu.VMEM((XR, S, D_EMB), jnp.float32),
            pltpu.SemaphoreType.DMA,
            pltpu.SemaphoreType.DMA,
            pltpu.SemaphoreType.DMA,
            pltpu.SemaphoreType.DMA,
            pltpu.SemaphoreType.DMA,
            pltpu.SemaphoreType.DMA,
        ],
    )
    out_p = run(x, table)
    return out_p[:, :, :64]
